# Initial kernel scaffold; baseline (speedup 1.0000x reference)
import jax
import jax.numpy as jnp
from jax.experimental import pallas as pl


def kernel(adj_indices, adj_values, user_emb, item_emb):
    raise NotImplementedError("write your pallas kernel here")


# speedup vs baseline: 2.0585x; 2.0585x over previous
import functools

import jax
import jax.numpy as jnp
from jax import lax
from jax.experimental import pallas as pl
from jax.experimental.pallas import tpu as pltpu
from jax.experimental.pallas import tpu_sc as plsc

N_USERS = 25000
N_ITEMS = 25000
N = N_USERS + N_ITEMS
D = 64
N_LAYERS = 3
E = 800000

NC = 2
NS = 16
QN = N // 4
AR = 12560
ZCH = AR // NS
WB = 781
K = 512
CHUNKS = 98
EPT = K * CHUNKS
E_PAD = NS * EPT


def _zero_rows(rows_v):
    def _z(i, _):
        z = jnp.zeros((16,), jnp.float32)
        for d in range(D // 16):
            rows_v[i, pl.ds(d * 16, 16)] = z
        return 0
    lax.fori_loop(0, K, _z, 0)


def _layer_body(x_hbm, row_hbm, col_hbm, val_hbm, y_hbm,
                colv, rowlocv, valv, rows_v, acc, sem):
    c = lax.axis_index("c")
    s = lax.axis_index("s")
    ebase = s * EPT

    for p in range(2):
        base_row = (c * 2 + p) * QN

        _zero_rows(rows_v)
        zbase = s * ZCH
        pltpu.sync_copy(rows_v.at[pl.ds(0, K)], acc.at[pl.ds(zbase, K)])
        pltpu.sync_copy(rows_v.at[pl.ds(0, ZCH - K)],
                        acc.at[pl.ds(zbase + K, ZCH - K)])
        plsc.subcore_barrier()

        def _chunk(i, _):
            eb = ebase + i * K
            pltpu.sync_copy(col_hbm.at[pl.ds(eb, K)], colv)
            pltpu.async_copy(x_hbm.at[colv], rows_v, sem).wait()
            pltpu.sync_copy(row_hbm.at[pl.ds(eb, K)], colv)
            pltpu.sync_copy(val_hbm.at[pl.ds(eb, K)], valv)

            def _loc(j, _):
                r = colv[pl.ds(j * 16, 16)]
                loc = r - base_row
                ok = (loc >= 0) & (loc < QN)
                rowlocv[pl.ds(j * 16, 16)] = jnp.where(ok, loc, QN)
                return 0
            lax.fori_loop(0, K // 16, _loc, 0)

            def _scale(j, _):
                vv = valv[pl.ds(j * 16, 16)]
                for l in range(16):
                    bv = jnp.broadcast_to(vv[l], (16,))
                    e = j * 16 + l
                    for d in range(D // 16):
                        sl = pl.ds(d * 16, 16)
                        rows_v[e, sl] = rows_v[e, sl] * bv
                return 0
            lax.fori_loop(0, K // 16, _scale, 0)

            pltpu.sync_copy(rows_v, acc.at[rowlocv], add=True)
            return 0

        lax.fori_loop(0, CHUNKS, _chunk, 0)
        plsc.subcore_barrier()

        wb = s * WB
        pltpu.sync_copy(acc.at[pl.ds(wb, WB)],
                        y_hbm.at[pl.ds(base_row + wb, WB)])

        @pl.when(s == 0)
        def _():
            pltpu.sync_copy(acc.at[pl.ds(NS * WB, QN - NS * WB)],
                            y_hbm.at[pl.ds(base_row + NS * WB, QN - NS * WB)])

        plsc.subcore_barrier()


_layer = functools.partial(
    pl.kernel,
    out_type=jax.ShapeDtypeStruct((N, D), jnp.float32),
    mesh=plsc.VectorSubcoreMesh(core_axis_name="c", subcore_axis_name="s"),
    compiler_params=pltpu.CompilerParams(use_tc_tiling_on_sc=False),
    scratch_types=[
        pltpu.VMEM((K,), jnp.int32),
        pltpu.VMEM((K,), jnp.int32),
        pltpu.VMEM((K,), jnp.float32),
        pltpu.VMEM((K, D), jnp.float32),
        pltpu.VMEM_SHARED((AR, D), jnp.float32),
        pltpu.SemaphoreType.DMA,
    ],
)(_layer_body)


def _mean_body(x0, x1, x2, x3, o):
    o[...] = (x0[...] + x1[...] + x2[...] + x3[...]) * 0.25


def _mean(x0, x1, x2, x3):
    blk = 400
    grid = N // blk
    spec = pl.BlockSpec((blk, D), lambda i: (i, 0))
    return pl.pallas_call(
        _mean_body,
        grid=(grid,),
        in_specs=[spec] * 4,
        out_specs=spec,
        out_shape=jax.ShapeDtypeStruct((N, D), jnp.float32),
    )(x0, x1, x2, x3)


def kernel(adj_indices, adj_values, user_emb, item_emb):
    row = adj_indices[0].astype(jnp.int32)
    col = adj_indices[1].astype(jnp.int32)
    val = adj_values.astype(jnp.float32)

    pad = E_PAD - E
    row = jnp.concatenate([row, jnp.full((pad,), N, jnp.int32)])
    col = jnp.concatenate([col, jnp.zeros((pad,), jnp.int32)])
    val = jnp.concatenate([val, jnp.zeros((pad,), jnp.float32)])

    x0 = jnp.concatenate([user_emb, item_emb], axis=0)
    xs = [x0]
    x = x0
    for _ in range(N_LAYERS):
        x = _layer(x, row, col, val)
        xs.append(x)

    out = _mean(*xs)
    return (out[:N_USERS], out[N_USERS:])
